# baseline (speedup 1.0000x reference)
import jax
import jax.numpy as jnp
from jax import lax
from jax.experimental import pallas as pl
from jax.experimental.pallas import tpu as pltpu
from jax.experimental.pallas import tpu_sc as plsc

_N_ROWS = 4096
_N_FIELDS = 26
_FIELD_SIZE = 1000
_ROW_WORDS = _N_FIELDS * _FIELD_SIZE

_TC_ROWS = 256
_SC_ROWS = _N_ROWS - _TC_ROWS

_NUM_CORES = 2
_NUM_SUBCORES = 16
_NUM_WORKERS = _NUM_CORES * _NUM_SUBCORES
_ROWS_PER_W = _SC_ROWS // _NUM_WORKERS
_CODES_PER_W = _ROWS_PER_W * _N_FIELDS
_STRIPE = 8
_NQ = 4
_CHUNK_W = 6400
_TAIL_START = _NQ * _CHUNK_W
_TAIL_W = _ROW_WORDS - _TAIL_START
_N_STRIPES_W = _ROWS_PER_W // _STRIPE

_TC_BLOCK = 128


def _sc_body(inp_ref, out_ref, b0, b1, t0, t1, codes, s0, s1, ts0, ts1):
    bufs = (b0, b1)
    sems = (s0, s1)
    tbufs = (t0, t1)
    tsems = (ts0, ts1)
    wid = lax.axis_index("s") * _NUM_CORES + lax.axis_index("c")
    code_off = (_TC_ROWS + wid * _ROWS_PER_W) * _N_FIELDS
    pltpu.sync_copy(inp_ref.at[pl.ds(code_off, _CODES_PER_W)], codes)

    zeros = jnp.zeros((16,), jnp.float32)
    ones = jnp.ones((16,), jnp.float32)
    iota = lax.iota(jnp.int32, 16)
    mask_hi = iota >= 6

    def zero_fill(bb, width):
        def zero_row(s, carry):
            def zero_body(i, carry2):
                bb[s, pl.ds(i * 16, 16)] = zeros
                return carry2

            return lax.fori_loop(0, width // 16, zero_body, carry)

        lax.fori_loop(0, _STRIPE, zero_row, 0)

    for b in range(2):
        zero_fill(bufs[b], _CHUNK_W)
        zero_fill(tbufs[b], _TAIL_W)

    row_base = _TC_ROWS + wid * _ROWS_PER_W

    def write_marks(bb, stripe_l, cstart, width, val):
        for s in range(_STRIPE):
            rl = stripe_l * _STRIPE + s
            c0 = codes[pl.ds(rl * _N_FIELDS, 16)].astype(jnp.int32)
            c1 = codes[pl.ds(rl * _N_FIELDS + 10, 16)].astype(jnp.int32)
            pos0 = iota * _FIELD_SIZE + c0 - cstart
            pos1 = (iota + 10) * _FIELD_SIZE + c1 - cstart
            m0 = (pos0 >= 0) & (pos0 < width)
            m1 = mask_hi & (pos1 >= 0) & (pos1 < width)
            svec = jnp.full((16,), s, jnp.int32)
            plsc.store_scatter(bb, [svec, pos0], val, mask=m0)
            plsc.store_scatter(bb, [svec, pos1], val, mask=m1)

    def chunk_dst(ci):
        stripe_l = ci // _NQ
        q = ci % _NQ
        row0 = row_base + stripe_l * _STRIPE
        return out_ref.at[
            pl.ds(row0, _STRIPE), pl.ds(q * _CHUNK_W, _CHUNK_W)
        ]

    def tail_dst(stripe_l):
        row0 = row_base + stripe_l * _STRIPE
        return out_ref.at[pl.ds(row0, _STRIPE), pl.ds(_TAIL_START, _TAIL_W)]

    n_chunks = _N_STRIPES_W * _NQ

    def group_body(g, carry):
        for b in range(2):
            bb = bufs[b]
            ci = g * 2 + b

            @pl.when(g > 0)
            def _(bb=bb, ci=ci, b=b):
                pltpu.make_async_copy(bb, chunk_dst(ci - 2), sems[b]).wait()
                oci = ci - 2
                write_marks(bb, oci // _NQ, (oci % _NQ) * _CHUNK_W,
                            _CHUNK_W, zeros)

            write_marks(bb, ci // _NQ, (ci % _NQ) * _CHUNK_W, _CHUNK_W, ones)
            pltpu.async_copy(bb, chunk_dst(ci), sems[b])
        return carry

    lax.fori_loop(0, n_chunks // 2, group_body, 0)

    def tail_body(g, carry):
        for b in range(2):
            tb = tbufs[b]
            stripe_l = g * 2 + b

            @pl.when(g > 0)
            def _(tb=tb, stripe_l=stripe_l, b=b):
                pltpu.make_async_copy(
                    tb, tail_dst(stripe_l - 2), tsems[b]
                ).wait()
                write_marks(tb, stripe_l - 2, _TAIL_START, _TAIL_W, zeros)

            @pl.when(stripe_l < _N_STRIPES_W)
            def _(tb=tb, stripe_l=stripe_l, b=b):
                write_marks(tb, stripe_l, _TAIL_START, _TAIL_W, ones)
                pltpu.async_copy(tb, tail_dst(stripe_l), tsems[b])
        return carry

    lax.fori_loop(0, (_N_STRIPES_W + 1) // 2, tail_body, 0)

    for b in range(2):
        pltpu.make_async_copy(
            bufs[b], chunk_dst(n_chunks - 2 + b), sems[b]
        ).wait()
    pltpu.make_async_copy(
        tbufs[0], tail_dst(_N_STRIPES_W - 1), tsems[0]
    ).wait()


def _tc_body(in_ref, alias_ref, out_ref):
    del alias_ref
    codes = in_ref[...]
    offs = jax.lax.broadcasted_iota(
        jnp.int32, (_TC_BLOCK, _FIELD_SIZE), 1
    ).astype(jnp.float32)
    for f in range(_N_FIELDS):
        out_ref[:, f * _FIELD_SIZE:(f + 1) * _FIELD_SIZE] = (
            offs == codes[:, f:f + 1]
        ).astype(jnp.float32)


def kernel(input):
    n = input.shape[0]
    flat_in = input.reshape(-1)
    mesh = plsc.VectorSubcoreMesh(
        core_axis_name="c", subcore_axis_name="s"
    )
    sc_out = pl.kernel(
        _sc_body,
        out_type=jax.ShapeDtypeStruct((n, _ROW_WORDS), jnp.float32),
        mesh=mesh,
        compiler_params=pltpu.CompilerParams(
            needs_layout_passes=False, use_tc_tiling_on_sc=True
        ),
        scratch_types=[
            pltpu.VMEM((_STRIPE, _CHUNK_W), jnp.float32),
            pltpu.VMEM((_STRIPE, _CHUNK_W), jnp.float32),
            pltpu.VMEM((_STRIPE, _TAIL_W), jnp.float32),
            pltpu.VMEM((_STRIPE, _TAIL_W), jnp.float32),
            pltpu.VMEM((_CODES_PER_W,), jnp.float32),
            pltpu.SemaphoreType.DMA,
            pltpu.SemaphoreType.DMA,
            pltpu.SemaphoreType.DMA,
            pltpu.SemaphoreType.DMA,
        ],
    )(flat_in)

    return pl.pallas_call(
        _tc_body,
        grid=(_TC_ROWS // _TC_BLOCK,),
        in_specs=[
            pl.BlockSpec((_TC_BLOCK, _N_FIELDS), lambda r: (r, 0)),
            pl.BlockSpec(memory_space=pltpu.MemorySpace.HBM),
        ],
        out_specs=pl.BlockSpec((_TC_BLOCK, _ROW_WORDS), lambda r: (r, 0)),
        out_shape=jax.ShapeDtypeStruct((n, _ROW_WORDS), jnp.float32),
        input_output_aliases={1: 0},
    )(input, sc_out)


# speedup vs baseline: 1.0843x; 1.0843x over previous
import jax
import jax.numpy as jnp
from jax.experimental import pallas as pl

_N_FIELDS = 26
_FIELD_SIZE = 1000
_ROW_WORDS = _N_FIELDS * _FIELD_SIZE
_ROW_BLOCK = 256


def _onehot_block(in_ref, out_ref):
    codes = in_ref[...]
    offs = jax.lax.broadcasted_iota(
        jnp.int32, (_ROW_BLOCK, _FIELD_SIZE), 1
    ).astype(jnp.float32)
    for f in range(_N_FIELDS):
        out_ref[:, f * _FIELD_SIZE:(f + 1) * _FIELD_SIZE] = (
            offs == codes[:, f:f + 1]
        ).astype(jnp.float32)


def kernel(input):
    n = input.shape[0]
    grid = (n // _ROW_BLOCK,)
    return pl.pallas_call(
        _onehot_block,
        grid=grid,
        in_specs=[pl.BlockSpec((_ROW_BLOCK, _N_FIELDS), lambda r: (r, 0))],
        out_specs=pl.BlockSpec((_ROW_BLOCK, _ROW_WORDS), lambda r: (r, 0)),
        out_shape=jax.ShapeDtypeStruct((n, _ROW_WORDS), jnp.float32),
    )(input)
